# Initial kernel scaffold; baseline (speedup 1.0000x reference)
import jax
import jax.numpy as jnp
from jax.experimental import pallas as pl


def kernel(x, edge_index, W1, W2):
    raise NotImplementedError("write your pallas kernel here")


# speedup vs baseline: 8.6539x; 8.6539x over previous
import functools

import jax
import jax.numpy as jnp
from jax import lax
from jax.experimental import pallas as pl
from jax.experimental.pallas import tpu as pltpu
from jax.experimental.pallas import tpu_sc as plsc


def _pick_chunk(epw):
    for ch in (128, 80, 64, 40, 32, 16, 8):
        if epw % ch == 0:
            return ch
    raise ValueError(f"edges per worker {epw} not divisible by 8")


def _memset_rows(ref, nrows, ncols, value):
    vec = jnp.full((16,), value, jnp.float32)

    def body(i, _):
        for j in range(ncols // 16):
            ref[i, pl.ds(j * 16, 16)] = vec
        return 0

    lax.fori_loop(0, nrows, body, 0)


def _zero_stripe(acc_sh, zeros_v, base_row, nrows, ch):
    nfull = nrows // ch
    rem = nrows - nfull * ch

    def body(i, _):
        pltpu.sync_copy(zeros_v, acc_sh.at[pl.ds(base_row + i * ch, ch)])
        return 0

    lax.fori_loop(0, nfull, body, 0)
    if rem:
        pltpu.sync_copy(zeros_v.at[pl.ds(0, rem)],
                        acc_sh.at[pl.ds(base_row + nfull * ch, rem)])


@functools.lru_cache(maxsize=None)
def _make_degree(n, e):
    info = plsc.get_sparse_core_info()
    nc, ns = info.num_cores, info.num_subcores
    nw = nc * ns
    epw = e // nw
    ch = _pick_chunk(epw)
    nchunk = epw // ch
    n_pad = pl.cdiv(n, ns * 8) * ns * 8
    rpt = n_pad // ns
    mesh = plsc.VectorSubcoreMesh(core_axis_name="c", subcore_axis_name="s")

    @functools.partial(
        pl.kernel,
        out_type=jax.ShapeDtypeStruct((nc, n_pad, 128), jnp.float32),
        mesh=mesh,
        scratch_types=[
            pltpu.VMEM((ch,), jnp.int32),
            pltpu.VMEM((ch, 128), jnp.float32),
            pltpu.VMEM((ch, 128), jnp.float32),
            pltpu.VMEM_SHARED((n_pad, 128), jnp.float32),
        ],
    )
    def kern(row_hbm, out_hbm, row_v, ones_v, zeros_v, acc_sh):
        cid = lax.axis_index("c")
        sid = lax.axis_index("s")
        wid = sid * nc + cid

        _memset_rows(ones_v, ch, 128, 1.0)
        _memset_rows(zeros_v, ch, 128, 0.0)
        base_row = sid * rpt
        _zero_stripe(acc_sh, zeros_v, base_row, rpt, ch)
        plsc.subcore_barrier()

        ebase = wid * epw

        def chunk(i, _):
            pltpu.sync_copy(row_hbm.at[pl.ds(ebase + i * ch, ch)], row_v)
            pltpu.sync_copy(ones_v, acc_sh.at[row_v], add=True)
            return 0

        lax.fori_loop(0, nchunk, chunk, 0)
        plsc.subcore_barrier()

        pltpu.sync_copy(acc_sh.at[pl.ds(base_row, rpt)],
                        out_hbm.at[cid, pl.ds(base_row, rpt)])

    return kern


@functools.lru_cache(maxsize=None)
def _make_edge_aggregate(n, e, d):
    info = plsc.get_sparse_core_info()
    nc, ns = info.num_cores, info.num_subcores
    nw = nc * ns
    epw = e // nw
    ch = _pick_chunk(epw)
    nchunk = epw // ch
    n_pad = pl.cdiv(n, ns * 8) * ns * 8
    rpt = n_pad // ns
    mesh = plsc.VectorSubcoreMesh(core_axis_name="c", subcore_axis_name="s")

    @functools.partial(
        pl.kernel,
        out_type=jax.ShapeDtypeStruct((nc, n_pad, d), jnp.float32),
        mesh=mesh,
        scratch_types=[
            pltpu.VMEM((ch,), jnp.int32),
            pltpu.VMEM((ch,), jnp.int32),
            pltpu.VMEM((ch, d), jnp.float32),
            pltpu.VMEM_SHARED((n_pad, d), jnp.float32),
            pltpu.SemaphoreType.DMA,
        ],
    )
    def kern(table_hbm, col_hbm, row_hbm, out_hbm,
             col_v, row_v, rows_v, acc_sh, sem):
        cid = lax.axis_index("c")
        sid = lax.axis_index("s")
        wid = sid * nc + cid

        _memset_rows(rows_v, ch, d, 0.0)
        base_row = sid * rpt
        _zero_stripe(acc_sh, rows_v, base_row, rpt, ch)
        plsc.subcore_barrier()

        ebase = wid * epw

        def chunk(i, _):
            off = ebase + i * ch
            pltpu.sync_copy(col_hbm.at[pl.ds(off, ch)], col_v)
            pltpu.async_copy(table_hbm.at[col_v], rows_v, sem).wait()
            pltpu.sync_copy(row_hbm.at[pl.ds(off, ch)], row_v)
            pltpu.sync_copy(rows_v, acc_sh.at[row_v], add=True)
            return 0

        lax.fori_loop(0, nchunk, chunk, 0)
        plsc.subcore_barrier()

        pltpu.sync_copy(acc_sh.at[pl.ds(base_row, rpt)],
                        out_hbm.at[cid, pl.ds(base_row, rpt)])

    return kern


_BLK = 1000


def _dvec(deg_ref):
    rs = deg_ref[0, :, 0:1] + deg_ref[1, :, 0:1]
    return jnp.where(rs > 0, lax.rsqrt(rs), 0.0)


def _scale_matmul_body(deg_ref, x_ref, w_ref, o_ref):
    o_ref[...] = _dvec(deg_ref) * jnp.dot(
        x_ref[...], w_ref[...], preferred_element_type=jnp.float32)


def _tc_scale_matmul(deg, x, w):
    n, f = x.shape
    h = w.shape[1]
    grid = n // _BLK
    return pl.pallas_call(
        _scale_matmul_body,
        grid=(grid,),
        in_specs=[
            pl.BlockSpec((2, _BLK, 128), lambda i: (0, i, 0)),
            pl.BlockSpec((_BLK, f), lambda i: (i, 0)),
            pl.BlockSpec((f, h), lambda i: (0, 0)),
        ],
        out_specs=pl.BlockSpec((_BLK, h), lambda i: (i, 0)),
        out_shape=jax.ShapeDtypeStruct((n, h), jnp.float32),
    )(deg, x, w)


def _relu_matmul_body(deg_ref, a_ref, w_ref, o_ref):
    dv = _dvec(deg_ref)
    hidden = jnp.maximum(dv * (a_ref[0] + a_ref[1]), 0.0)
    o_ref[...] = dv * jnp.dot(
        hidden, w_ref[...], preferred_element_type=jnp.float32)


def _tc_relu_matmul(deg, agg, w, n):
    f = agg.shape[2]
    h = w.shape[1]
    grid = n // _BLK
    return pl.pallas_call(
        _relu_matmul_body,
        grid=(grid,),
        in_specs=[
            pl.BlockSpec((2, _BLK, 128), lambda i: (0, i, 0)),
            pl.BlockSpec((2, _BLK, f), lambda i: (0, i, 0)),
            pl.BlockSpec((f, h), lambda i: (0, 0)),
        ],
        out_specs=pl.BlockSpec((_BLK, h), lambda i: (i, 0)),
        out_shape=jax.ShapeDtypeStruct((n, h), jnp.float32),
    )(deg, agg, w)


def _encode_body(deg_ref, a_ref, o_ref):
    o_ref[...] = _dvec(deg_ref) * (a_ref[0] + a_ref[1])


def _tc_encode(deg, agg, n):
    c = agg.shape[2]
    grid = n // _BLK
    return pl.pallas_call(
        _encode_body,
        grid=(grid,),
        in_specs=[
            pl.BlockSpec((2, _BLK, 128), lambda i: (0, i, 0)),
            pl.BlockSpec((2, _BLK, c), lambda i: (0, i, 0)),
        ],
        out_specs=pl.BlockSpec((_BLK, c), lambda i: (i, 0)),
        out_shape=jax.ShapeDtypeStruct((n, c), jnp.float32),
    )(deg, agg)


def _decode_body(zi_ref, zj_ref, o_ref):
    zz = lax.dot_general(
        zi_ref[...], zj_ref[...], (((1,), (1,)), ((), ())),
        preferred_element_type=jnp.float32)
    o_ref[...] = 1.0 / (1.0 + jnp.exp(-zz))


def _tc_decode(z):
    n, c = z.shape
    blk = 1024
    grid = pl.cdiv(n, blk)
    return pl.pallas_call(
        _decode_body,
        grid=(grid, grid),
        in_specs=[
            pl.BlockSpec((blk, c), lambda i, j: (i, 0)),
            pl.BlockSpec((blk, c), lambda i, j: (j, 0)),
        ],
        out_specs=pl.BlockSpec((blk, blk), lambda i, j: (i, j)),
        out_shape=jax.ShapeDtypeStruct((n, n), jnp.float32),
    )(z, z)


def kernel(x, edge_index, W1, W2):
    n, _ = x.shape
    e = edge_index.shape[1]
    row = edge_index[0]
    col = edge_index[1]
    if W2.shape[1] % 128:
        W2 = jnp.pad(W2, ((0, 0), (0, 128 - W2.shape[1] % 128)))

    deg = _make_degree(n, e)(row)
    xf1 = _tc_scale_matmul(deg, x, W1)
    agg1 = _make_edge_aggregate(n, e, W1.shape[1])(xf1, col, row)
    xf2 = _tc_relu_matmul(deg, agg1, W2, n)
    agg2 = _make_edge_aggregate(n, e, W2.shape[1])(xf2, col, row)
    z = _tc_encode(deg, agg2, n)
    return _tc_decode(z)


# speedup vs baseline: 16.8746x; 1.9499x over previous
import functools

import jax
import jax.numpy as jnp
from jax import lax
from jax.experimental import pallas as pl
from jax.experimental.pallas import tpu as pltpu
from jax.experimental.pallas import tpu_sc as plsc


def _pick_chunk(epw):
    for ch in (128, 80, 64, 40, 32, 16, 8):
        if epw % ch == 0:
            return ch
    raise ValueError(f"edges per worker {epw} not divisible by 8")


def _memset_rows(ref, nrows, ncols, value):
    vec = jnp.full((16,), value, jnp.float32)

    def body(i, _):
        for j in range(ncols // 16):
            ref[i, pl.ds(j * 16, 16)] = vec
        return 0

    lax.fori_loop(0, nrows, body, 0)


def _memset_vec(ref, length, value):
    vec = jnp.full((16,), value, jnp.float32)
    for k in range(length // 16):
        ref[pl.ds(k * 16, 16)] = vec


def _copy_idx(src, dst, base, ch):
    for k in range(ch // 16):
        dst[pl.ds(k * 16, 16)] = src[pl.ds(base + k * 16, 16)]


def _zero_stripe(acc_sh, zeros_v, base_row, nrows, ch):
    nfull = nrows // ch
    rem = nrows - nfull * ch

    def body(i, _):
        pltpu.sync_copy(zeros_v, acc_sh.at[pl.ds(base_row + i * ch, ch)])
        return 0

    lax.fori_loop(0, nfull, body, 0)
    if rem:
        pltpu.sync_copy(zeros_v.at[pl.ds(0, rem)],
                        acc_sh.at[pl.ds(base_row + nfull * ch, rem)])


@functools.lru_cache(maxsize=None)
def _make_degree(n, e):
    info = plsc.get_sparse_core_info()
    nc, ns = info.num_cores, info.num_subcores
    nw = nc * ns
    epw = e // nw
    ch = _pick_chunk(epw)
    nchunk = epw // ch
    n_pad = pl.cdiv(n, ns * 128) * ns * 128
    rpt = n_pad // ns
    mesh = plsc.VectorSubcoreMesh(core_axis_name="c", subcore_axis_name="s")

    @functools.partial(
        pl.kernel,
        out_type=jax.ShapeDtypeStruct((nc, n_pad), jnp.float32),
        mesh=mesh,
        scratch_types=[
            pltpu.VMEM((epw,), jnp.int32),
            pltpu.VMEM((ch,), jnp.int32),
            pltpu.VMEM((ch,), jnp.float32),
            pltpu.VMEM((ch,), jnp.float32),
            pltpu.VMEM_SHARED((n_pad,), jnp.float32),
        ],
    )
    def kern(row_hbm, out_hbm, row_all, row_v, ones_v, zeros_v, acc_sh):
        cid = lax.axis_index("c")
        sid = lax.axis_index("s")
        wid = sid * nc + cid

        _memset_vec(ones_v, ch, 1.0)
        _memset_vec(zeros_v, ch, 0.0)
        base_row = sid * rpt

        def zbody(i, _):
            pltpu.sync_copy(zeros_v, acc_sh.at[pl.ds(base_row + i * ch, ch)])
            return 0

        lax.fori_loop(0, rpt // ch, zbody, 0)
        if rpt % ch:
            pltpu.sync_copy(zeros_v.at[pl.ds(0, rpt % ch)],
                            acc_sh.at[pl.ds(base_row + (rpt // ch) * ch,
                                            rpt % ch)])
        pltpu.sync_copy(row_hbm.at[pl.ds(wid * epw, epw)], row_all)
        plsc.subcore_barrier()

        def chunk(i, _):
            _copy_idx(row_all, row_v, i * ch, ch)
            pltpu.sync_copy(ones_v, acc_sh.at[row_v], add=True)
            return 0

        lax.fori_loop(0, nchunk, chunk, 0)
        plsc.subcore_barrier()

        pltpu.sync_copy(acc_sh.at[pl.ds(base_row, rpt)],
                        out_hbm.at[cid, pl.ds(base_row, rpt)])

    return kern


@functools.lru_cache(maxsize=None)
def _make_edge_aggregate(n, e, d):
    info = plsc.get_sparse_core_info()
    nc, ns = info.num_cores, info.num_subcores
    nw = nc * ns
    epw = e // nw
    ch = _pick_chunk(epw)
    nchunk = epw // ch
    n_pad = pl.cdiv(n, ns * 8) * ns * 8
    rpt = n_pad // ns
    mesh = plsc.VectorSubcoreMesh(core_axis_name="c", subcore_axis_name="s")

    @functools.partial(
        pl.kernel,
        out_type=jax.ShapeDtypeStruct((nc, n_pad, d), jnp.float32),
        mesh=mesh,
        scratch_types=[
            pltpu.VMEM((epw,), jnp.int32),
            pltpu.VMEM((epw,), jnp.int32),
            pltpu.VMEM((ch,), jnp.int32),
            pltpu.VMEM((ch,), jnp.int32),
            pltpu.VMEM((ch,), jnp.int32),
            pltpu.VMEM((ch, d), jnp.float32),
            pltpu.VMEM((ch, d), jnp.float32),
            pltpu.VMEM_SHARED((n_pad, d), jnp.float32),
            pltpu.SemaphoreType.DMA,
            pltpu.SemaphoreType.DMA,
        ],
    )
    def kern(table_hbm, col_hbm, row_hbm, out_hbm,
             col_all, row_all, col_v0, col_v1, row_v, rows_v0, rows_v1,
             acc_sh, sem0, sem1):
        cid = lax.axis_index("c")
        sid = lax.axis_index("s")
        wid = sid * nc + cid

        _memset_rows(rows_v0, ch, d, 0.0)
        base_row = sid * rpt
        _zero_stripe(acc_sh, rows_v0, base_row, rpt, ch)
        ebase = wid * epw
        pltpu.sync_copy(col_hbm.at[pl.ds(ebase, epw)], col_all)
        pltpu.sync_copy(row_hbm.at[pl.ds(ebase, epw)], row_all)
        plsc.subcore_barrier()

        def gather(i, col_v, rows_v, sem):
            _copy_idx(col_all, col_v, i * ch, ch)
            pltpu.async_copy(table_hbm.at[col_v], rows_v, sem)

        def wait_gather(rows_v, sem):
            pltpu.make_async_copy(table_hbm.at[pl.ds(0, ch)], rows_v,
                                  sem).wait()

        def scatter(i, rows_v):
            _copy_idx(row_all, row_v, i * ch, ch)
            pltpu.sync_copy(rows_v, acc_sh.at[row_v], add=True)

        if nchunk % 2:
            gather(0, col_v0, rows_v0, sem0)

            def step(g, _):
                i0 = g * 2
                gather(i0 + 1, col_v1, rows_v1, sem1)
                wait_gather(rows_v0, sem0)
                scatter(i0, rows_v0)
                gather(i0 + 2, col_v0, rows_v0, sem0)
                wait_gather(rows_v1, sem1)
                scatter(i0 + 1, rows_v1)
                return 0

            lax.fori_loop(0, (nchunk - 1) // 2, step, 0)
            wait_gather(rows_v0, sem0)
            scatter(nchunk - 1, rows_v0)
        else:
            def step_seq(i, _):
                gather(i, col_v0, rows_v0, sem0)
                wait_gather(rows_v0, sem0)
                scatter(i, rows_v0)
                return 0

            lax.fori_loop(0, nchunk, step_seq, 0)
        plsc.subcore_barrier()

        pltpu.sync_copy(acc_sh.at[pl.ds(base_row, rpt)],
                        out_hbm.at[cid, pl.ds(base_row, rpt)])

    return kern


_BLK = 1000


def _dvec(deg_ref):
    rs = deg_ref[:, 0:1] + deg_ref[:, 1:2]
    return jnp.where(rs > 0, lax.rsqrt(rs), 0.0)


def _scale_matmul_body(deg_ref, x_ref, w_ref, o_ref):
    o_ref[...] = _dvec(deg_ref) * jnp.dot(
        x_ref[...], w_ref[...], preferred_element_type=jnp.float32)


def _tc_scale_matmul(deg, x, w):
    n, f = x.shape
    h = w.shape[1]
    grid = n // _BLK
    return pl.pallas_call(
        _scale_matmul_body,
        grid=(grid,),
        in_specs=[
            pl.BlockSpec((_BLK, 2), lambda i: (i, 0)),
            pl.BlockSpec((_BLK, f), lambda i: (i, 0)),
            pl.BlockSpec((f, h), lambda i: (0, 0)),
        ],
        out_specs=pl.BlockSpec((_BLK, h), lambda i: (i, 0)),
        out_shape=jax.ShapeDtypeStruct((n, h), jnp.float32),
    )(deg, x, w)


def _relu_matmul_body(deg_ref, a_ref, w_ref, o_ref):
    dv = _dvec(deg_ref)
    hidden = jnp.maximum(dv * (a_ref[0] + a_ref[1]), 0.0)
    o_ref[...] = dv * jnp.dot(
        hidden, w_ref[...], preferred_element_type=jnp.float32)


def _tc_relu_matmul(deg, agg, w, n):
    f = agg.shape[2]
    h = w.shape[1]
    grid = n // _BLK
    return pl.pallas_call(
        _relu_matmul_body,
        grid=(grid,),
        in_specs=[
            pl.BlockSpec((_BLK, 2), lambda i: (i, 0)),
            pl.BlockSpec((2, _BLK, f), lambda i: (0, i, 0)),
            pl.BlockSpec((f, h), lambda i: (0, 0)),
        ],
        out_specs=pl.BlockSpec((_BLK, h), lambda i: (i, 0)),
        out_shape=jax.ShapeDtypeStruct((n, h), jnp.float32),
    )(deg, agg, w)


def _encode_body(deg_ref, a_ref, o_ref):
    o_ref[...] = _dvec(deg_ref) * (a_ref[0] + a_ref[1])


def _tc_encode(deg, agg, n):
    c = agg.shape[2]
    grid = n // _BLK
    return pl.pallas_call(
        _encode_body,
        grid=(grid,),
        in_specs=[
            pl.BlockSpec((_BLK, 2), lambda i: (i, 0)),
            pl.BlockSpec((2, _BLK, c), lambda i: (0, i, 0)),
        ],
        out_specs=pl.BlockSpec((_BLK, c), lambda i: (i, 0)),
        out_shape=jax.ShapeDtypeStruct((n, c), jnp.float32),
    )(deg, agg)


def _decode_body(zi_ref, zj_ref, o_ref):
    zz = lax.dot_general(
        zi_ref[...], zj_ref[...], (((1,), (1,)), ((), ())),
        preferred_element_type=jnp.float32)
    o_ref[...] = 1.0 / (1.0 + jnp.exp(-zz))


def _tc_decode(z):
    n, c = z.shape
    blk = 1024
    grid = pl.cdiv(n, blk)
    return pl.pallas_call(
        _decode_body,
        grid=(grid, grid),
        in_specs=[
            pl.BlockSpec((blk, c), lambda i, j: (i, 0)),
            pl.BlockSpec((blk, c), lambda i, j: (j, 0)),
        ],
        out_specs=pl.BlockSpec((blk, blk), lambda i, j: (i, j)),
        out_shape=jax.ShapeDtypeStruct((n, n), jnp.float32),
    )(z, z)


def kernel(x, edge_index, W1, W2):
    n, _ = x.shape
    e = edge_index.shape[1]
    row = edge_index[0]
    col = edge_index[1]
    if W2.shape[1] % 128:
        W2 = jnp.pad(W2, ((0, 0), (0, 128 - W2.shape[1] % 128)))

    deg = _make_degree(n, e)(row).T
    xf1 = _tc_scale_matmul(deg, x, W1)
    agg1 = _make_edge_aggregate(n, e, W1.shape[1])(xf1, col, row)
    xf2 = _tc_relu_matmul(deg, agg1, W2, n)
    agg2 = _make_edge_aggregate(n, e, W2.shape[1])(xf2, col, row)
    z = _tc_encode(deg, agg2, n)
    return _tc_decode(z)
